# baseline (speedup 1.0000x reference)
import functools

import jax
import jax.numpy as jnp
from jax import lax
from jax.experimental import pallas as pl
from jax.experimental.pallas import tpu as pltpu
from jax.experimental.pallas import tpu_sc as plsc

NIN = 10
NOUT = 128
NDISC = 3
NHID = NDISC ** NIN
HALF = 29568
NBLK = HALF // 128
BATCH = 1024

NC = 2
NS = 16
NW = NC * NS
B_PER_W = BATCH // NW
L = 16

_POW3 = [NDISC ** i for i in range(NIN)]

_mesh = plsc.VectorSubcoreMesh(
    core_axis_name="c", subcore_axis_name="s", num_cores=NC, num_subcores=NS
)


@functools.partial(
    pl.kernel,
    out_type=jax.ShapeDtypeStruct((BATCH,), jnp.int32),
    mesh=_mesh,
    scratch_types=[
        pltpu.VMEM((NIN * B_PER_W,), jnp.float32),
        pltpu.VMEM((B_PER_W,), jnp.int32),
    ],
)
def _sc_bucketize(xr_hbm, idx_hbm, xv, idx_v):
    wid = lax.axis_index("s") * NC + lax.axis_index("c")
    base = wid * B_PER_W
    pltpu.sync_copy(xr_hbm.at[pl.ds(wid * (NIN * B_PER_W), NIN * B_PER_W)], xv)
    neg = jnp.full((L,), -0.1, jnp.float32)
    pos = jnp.full((L,), 0.1, jnp.float32)
    for ch in range(B_PER_W // L):
        acc = jnp.zeros((L,), jnp.int32)
        for i in range(NIN):
            v = xv[pl.ds(i * B_PER_W + ch * L, L)]
            p3 = jnp.full((L,), _POW3[i], jnp.int32)
            zero = jnp.zeros((L,), jnp.int32)
            d = jnp.where(v > neg, p3, zero) + jnp.where(v > pos, p3, zero)
            acc = acc + d
        idx_v[pl.ds(ch * L, L)] = acc
    pltpu.sync_copy(idx_v, idx_hbm.at[pl.ds(base, B_PER_W)])


@functools.partial(
    pl.kernel,
    out_type=jax.ShapeDtypeStruct((BATCH, NOUT), jnp.float32),
    mesh=_mesh,
    scratch_types=[
        pltpu.VMEM((B_PER_W,), jnp.int32),
        pltpu.VMEM((B_PER_W,), jnp.int32),
        pltpu.VMEM((B_PER_W,), jnp.int32),
        pltpu.VMEM((B_PER_W, NOUT), jnp.int32),
        pltpu.VMEM((B_PER_W, NOUT), jnp.float32),
        pltpu.SemaphoreType.DMA,
    ],
)
def _sc_gather(idx_hbm, P_hbm, out_hbm, idx_v, row_v, flag_v, pk_v, rows_f, sem):
    wid = lax.axis_index("s") * NC + lax.axis_index("c")
    base = wid * B_PER_W
    pltpu.sync_copy(idx_hbm.at[pl.ds(base, B_PER_W)], idx_v)
    zero = jnp.zeros((L,), jnp.int32)
    cth = jnp.full((L,), HALF - 1, jnp.int32)
    cneg = jnp.full((L,), -HALF, jnp.int32)
    onei = jnp.full((L,), 1, jnp.int32)
    for ch in range(B_PER_W // L):
        iv = idx_v[pl.ds(ch * L, L)]
        row_v[pl.ds(ch * L, L)] = iv + jnp.where(iv > cth, cneg, zero)
        flag_v[pl.ds(ch * L, L)] = jnp.where(iv > cth, onei, zero)
    pltpu.async_copy(P_hbm.at[row_v], pk_v, sem).wait()
    c64k = jnp.full((L,), 65536, jnp.int32)
    cm65535 = jnp.full((L,), -65535, jnp.int32)
    for ch in range(B_PER_W // L):
        fv = flag_v[pl.ds(ch * L, L)]
        for k in range(L):
            b = ch * L + k
            factor = c64k + jnp.broadcast_to(fv[k], (L,)) * cm65535
            for oc in range(NOUT // L):
                w = pk_v[b, pl.ds(oc * L, L)]
                bits = w * factor
                rows_f[b, pl.ds(oc * L, L)] = lax.bitcast_convert_type(bits, jnp.float32)
    pltpu.sync_copy(rows_f, out_hbm.at[pl.ds(base, B_PER_W)])


def _tc_pack_body(wlo_ref, whi_ref, p_ref):
    pk = pltpu.pack_elementwise(
        [wlo_ref[...], whi_ref[...]], packed_dtype=jnp.bfloat16)
    p_ref[...] = lax.bitcast_convert_type(pk, jnp.int32).T


PBLK = 896
_tc_pack = pl.pallas_call(
    _tc_pack_body,
    grid=(HALF // PBLK,),
    in_specs=[
        pl.BlockSpec((NOUT, PBLK), lambda j: (0, j)),
        pl.BlockSpec((NOUT, PBLK), lambda j: (0, j + HALF // PBLK)),
    ],
    out_specs=pl.BlockSpec((PBLK, NOUT), lambda j: (j, 0)),
    out_shape=jax.ShapeDtypeStruct((HALF, NOUT), jnp.int32),
)


def kernel(x, W):
    xr = x.reshape(NW, B_PER_W, NIN).transpose(0, 2, 1).reshape(-1)
    idx = _sc_bucketize(xr)
    P = _tc_pack(W, W)
    return _sc_gather(idx, P)


# speedup vs baseline: 3.6900x; 3.6900x over previous
import functools

import jax
import jax.numpy as jnp
from jax import lax
from jax.experimental import pallas as pl
from jax.experimental.pallas import tpu as pltpu
from jax.experimental.pallas import tpu_sc as plsc

NIN = 10
NOUT = 128
NDISC = 3
NHID = NDISC ** NIN
BATCH = 1024

NC = 2
NS = 16
NW = NC * NS
B_PER_W = BATCH // NW
L = 16

_POW3 = [NDISC ** i for i in range(NIN)]

_mesh = plsc.VectorSubcoreMesh(
    core_axis_name="c", subcore_axis_name="s", num_cores=NC, num_subcores=NS
)


@functools.partial(
    pl.kernel,
    out_type=jax.ShapeDtypeStruct((BATCH, NOUT), jnp.float32),
    mesh=_mesh,
    scratch_types=[
        pltpu.VMEM((NIN * B_PER_W,), jnp.float32),
        pltpu.VMEM((B_PER_W,), jnp.int32),
        pltpu.VMEM((B_PER_W, NOUT), jnp.float32),
        pltpu.SemaphoreType.DMA,
    ],
)
def _sc_lookup(xr_hbm, WT_hbm, out_hbm, xv, idx_v, rows_v, sem):
    wid = lax.axis_index("s") * NC + lax.axis_index("c")
    base = wid * B_PER_W
    pltpu.sync_copy(xr_hbm.at[pl.ds(wid * (NIN * B_PER_W), NIN * B_PER_W)], xv)
    neg = jnp.full((L,), -0.1, jnp.float32)
    pos = jnp.full((L,), 0.1, jnp.float32)
    for ch in range(B_PER_W // L):
        acc = jnp.zeros((L,), jnp.int32)
        for i in range(NIN):
            v = xv[pl.ds(i * B_PER_W + ch * L, L)]
            p3 = jnp.full((L,), _POW3[i], jnp.int32)
            zero = jnp.zeros((L,), jnp.int32)
            d = jnp.where(v > neg, p3, zero) + jnp.where(v > pos, p3, zero)
            acc = acc + d
        idx_v[pl.ds(ch * L, L)] = acc
    pltpu.async_copy(WT_hbm.at[idx_v], rows_v, sem).wait()
    pltpu.sync_copy(rows_v, out_hbm.at[pl.ds(base, B_PER_W)])


def kernel(x, W):
    xr = x.reshape(NW, B_PER_W, NIN).transpose(0, 2, 1).reshape(-1)
    WT = W.T
    return _sc_lookup(xr, WT)
